# baseline (speedup 1.0000x reference)
import jax
import jax.numpy as jnp
from jax.experimental import pallas as pl

_B, _A, _C, _M = 4, 65536, 80, 32
_ALPHA, _GAMMA, _EPS = 0.25, 2.0, 1e-4
_BA = 2048
_NA = _A // _BA


def _focal_kernel(cls_ref, reg_ref, anc_ref, ann_ref, out_ref):
    ai = pl.program_id(1)

    @pl.when(ai == 0)
    def _init():
        out_ref[...] = jnp.zeros_like(out_ref)

    anc = anc_ref[0]
    ax1 = anc[:, 0]
    ay1 = anc[:, 1]
    ax2 = anc[:, 2]
    ay2 = anc[:, 3]
    ann = ann_ref[0]
    bx1 = ann[:, 0][None, :]
    by1 = ann[:, 1][None, :]
    bx2 = ann[:, 2][None, :]
    by2 = ann[:, 3][None, :]
    blab = ann[:, 4][None, :]

    iw = jnp.minimum(ax2[:, None], bx2) - jnp.maximum(ax1[:, None], bx1)
    ih = jnp.minimum(ay2[:, None], by2) - jnp.maximum(ay1[:, None], by1)
    iw = jnp.clip(iw, 0.0, None)
    ih = jnp.clip(ih, 0.0, None)
    area_a = ((ax2 - ax1) * (ay2 - ay1))[:, None]
    area_b = (bx2 - bx1) * (by2 - by1)
    ua = jnp.clip(area_a + area_b - iw * ih, 1e-8, None)
    iou = iw * ih / ua
    iou_max = jnp.max(iou, axis=1)
    midx = jax.lax.broadcasted_iota(jnp.int32, (_BA, _M), 1)
    arg = jnp.min(jnp.where(iou == iou_max[:, None], midx, _M), axis=1)

    pos = iou_max > 0.1
    neg = iou_max <= 0.01
    posf = pos.astype(jnp.float32)
    valf = jnp.logical_or(pos, neg).astype(jnp.float32)

    ohm = (midx == arg[:, None]).astype(jnp.float32)
    gx1 = jnp.sum(ohm * bx1, axis=1)
    gy1 = jnp.sum(ohm * by1, axis=1)
    gx2 = jnp.sum(ohm * bx2, axis=1)
    gy2 = jnp.sum(ohm * by2, axis=1)
    glab = jnp.sum(ohm * blab, axis=1)

    cls = jnp.clip(cls_ref[0], _EPS, 1.0 - _EPS)
    term0 = (1.0 - _ALPHA) * (cls * cls) * (-jnp.log(1.0 - cls))
    s = jnp.sum(term0, axis=1)

    cidx = jax.lax.broadcasted_iota(jnp.int32, (_BA, _C), 1)
    ohc = (cidx == glab.astype(jnp.int32)[:, None]).astype(jnp.float32)
    cls_sel = jnp.sum(ohc * cls, axis=1)
    t0_sel = (1.0 - _ALPHA) * (cls_sel * cls_sel) * (-jnp.log(1.0 - cls_sel))
    t1_sel = _ALPHA * ((1.0 - cls_sel) * (1.0 - cls_sel)) * (-jnp.log(cls_sel))
    clf_part = jnp.sum(valf * s + posf * (t1_sel - t0_sel))
    npos_part = jnp.sum(posf)

    aw = ax2 - ax1
    ah = ay2 - ay1
    acx = ax1 + 0.5 * aw
    acy = ay1 + 0.5 * ah
    gwr = gx2 - gx1
    ghr = gy2 - gy1
    gcx = gx1 + 0.5 * gwr
    gcy = gy1 + 0.5 * ghr
    gw = jnp.clip(gwr, 1.0, None)
    gh = jnp.clip(ghr, 1.0, None)
    tdx = ((gcx - acx) / aw) / 0.1
    tdy = ((gcy - acy) / ah) / 0.1
    tdw = jnp.log(gw / aw) / 0.2
    tdh = jnp.log(gh / ah) / 0.2
    regb = reg_ref[0]
    rl = jnp.zeros((_BA,), jnp.float32)
    for k, tk in enumerate((tdx, tdy, tdw, tdh)):
        d = jnp.abs(tk - regb[:, k])
        rl = rl + jnp.where(d < 1.0 / 9.0, 0.5 * 9.0 * (d * d), d - 0.5 / 9.0)
    reg_part = jnp.sum(posf * rl)

    lane = jax.lax.broadcasted_iota(jnp.int32, (1, 1, 128), 2)
    vec = (jnp.where(lane == 0, clf_part, 0.0)
           + jnp.where(lane == 1, reg_part, 0.0)
           + jnp.where(lane == 2, npos_part, 0.0))
    out_ref[...] += vec


def kernel(classifications, regressions, anchors, annotations):
    sums = pl.pallas_call(
        _focal_kernel,
        grid=(_B, _NA),
        in_specs=[
            pl.BlockSpec((1, _BA, _C), lambda b, a: (b, a, 0)),
            pl.BlockSpec((1, _BA, 4), lambda b, a: (b, a, 0)),
            pl.BlockSpec((1, _BA, 4), lambda b, a: (0, a, 0)),
            pl.BlockSpec((1, _M, 5), lambda b, a: (b, 0, 0)),
        ],
        out_specs=pl.BlockSpec((1, 1, 128), lambda b, a: (b, 0, 0)),
        out_shape=jax.ShapeDtypeStruct((_B, 1, 128), jnp.float32),
    )(classifications, regressions, anchors, annotations)
    clf_sum = sums[:, 0, 0]
    reg_sum = sums[:, 0, 1]
    npos = sums[:, 0, 2]
    clf = clf_sum / jnp.clip(npos, 1.0, None)
    reg = reg_sum / jnp.clip(npos * 4.0, 1.0, None)
    return jnp.concatenate([jnp.mean(clf, keepdims=True),
                            jnp.mean(reg, keepdims=True)])


# speedup vs baseline: 4.9491x; 4.9491x over previous
import jax
import jax.numpy as jnp
from jax.experimental import pallas as pl
from jax.experimental.pallas import tpu as pltpu

_B, _A, _C, _M = 4, 65536, 80, 32
_ALPHA, _EPS = 0.25, 1e-4
_BA = 2048
_NA = _A // _BA
_LR = _BA // 128


def _focal_kernel(ann_ref, cls_ref, reg_ref, anc_ref, out_ref):
    bi = pl.program_id(0)
    ai = pl.program_id(1)

    @pl.when(ai == 0)
    def _init():
        out_ref[...] = jnp.zeros_like(out_ref)

    ax1 = anc_ref[0]
    ay1 = anc_ref[1]
    ax2 = anc_ref[2]
    ay2 = anc_ref[3]
    area_a = (ax2 - ax1) * (ay2 - ay1)

    def iou_of(m):
        bx1 = ann_ref[bi, m, 0]
        by1 = ann_ref[bi, m, 1]
        bx2 = ann_ref[bi, m, 2]
        by2 = ann_ref[bi, m, 3]
        area_b = ann_ref[bi, m, 4]
        iw = jnp.clip(jnp.minimum(ax2, bx2) - jnp.maximum(ax1, bx1), 0.0, None)
        ih = jnp.clip(jnp.minimum(ay2, by2) - jnp.maximum(ay1, by1), 0.0, None)
        inter = iw * ih
        ua = jnp.clip(area_a + area_b - inter, 1e-8, None)
        return inter / ua

    best = iou_of(0)
    gx1 = jnp.full_like(best, ann_ref[bi, 0, 0])
    gy1 = jnp.full_like(best, ann_ref[bi, 0, 1])
    gx2 = jnp.full_like(best, ann_ref[bi, 0, 2])
    gy2 = jnp.full_like(best, ann_ref[bi, 0, 3])
    glab = jnp.full_like(best, ann_ref[bi, 0, 5])
    for m in range(1, _M):
        iou_m = iou_of(m)
        upd = iou_m > best
        best = jnp.where(upd, iou_m, best)
        gx1 = jnp.where(upd, ann_ref[bi, m, 0], gx1)
        gy1 = jnp.where(upd, ann_ref[bi, m, 1], gy1)
        gx2 = jnp.where(upd, ann_ref[bi, m, 2], gx2)
        gy2 = jnp.where(upd, ann_ref[bi, m, 3], gy2)
        glab = jnp.where(upd, ann_ref[bi, m, 5], glab)

    pos = best > 0.1
    posf = pos.astype(jnp.float32)
    valf = jnp.where(best <= 0.01, 1.0, posf)

    aw = ax2 - ax1
    ah = ay2 - ay1
    acx = ax1 + 0.5 * aw
    acy = ay1 + 0.5 * ah
    gwr = gx2 - gx1
    ghr = gy2 - gy1
    gcx = gx1 + 0.5 * gwr
    gcy = gy1 + 0.5 * ghr
    gw = jnp.clip(gwr, 1.0, None)
    gh = jnp.clip(ghr, 1.0, None)
    tdx = ((gcx - acx) / aw) / 0.1
    tdy = ((gcy - acy) / ah) / 0.1
    tdw = jnp.log(gw / aw) / 0.2
    tdh = jnp.log(gh / ah) / 0.2
    rl = jnp.zeros_like(best)
    for k, tk in enumerate((tdx, tdy, tdw, tdh)):
        d = jnp.abs(tk - reg_ref[0, k])
        rl = rl + jnp.where(d < 1.0 / 9.0, 0.5 * 9.0 * (d * d), d - 0.5 / 9.0)
    reg_part = jnp.sum(posf * rl)
    npos_part = jnp.sum(posf)

    cls_t = jnp.clip(cls_ref[0].T, _EPS, 1.0 - _EPS)
    term0 = (1.0 - _ALPHA) * (cls_t * cls_t) * (-jnp.log(1.0 - cls_t))
    s_row = jnp.sum(term0, axis=0)

    glab_row = jnp.reshape(glab, (1, _BA)).astype(jnp.int32)
    sub = jax.lax.broadcasted_iota(jnp.int32, (_C, _BA), 0)
    cls_sel = jnp.sum(jnp.where(sub == glab_row, cls_t, 0.0), axis=0)
    t0s = (1.0 - _ALPHA) * (cls_sel * cls_sel) * (-jnp.log(1.0 - cls_sel))
    t1s = _ALPHA * ((1.0 - cls_sel) * (1.0 - cls_sel)) * (-jnp.log(cls_sel))
    valf_row = jnp.reshape(valf, (_BA,))
    posf_row = jnp.reshape(posf, (_BA,))
    clf_part = jnp.sum(valf_row * s_row + posf_row * (t1s - t0s))

    lane = jax.lax.broadcasted_iota(jnp.int32, (1, 1, 128), 2)
    vec = (jnp.where(lane == 0, clf_part, 0.0)
           + jnp.where(lane == 1, reg_part, 0.0)
           + jnp.where(lane == 2, npos_part, 0.0))
    out_ref[...] += vec


def kernel(classifications, regressions, anchors, annotations):
    anc = jnp.transpose(anchors[0]).reshape(4, _A // 128, 128)
    reg = jnp.transpose(regressions, (0, 2, 1)).reshape(_B, 4, _A // 128, 128)
    area_b = ((annotations[:, :, 2] - annotations[:, :, 0])
              * (annotations[:, :, 3] - annotations[:, :, 1]))
    ann = jnp.concatenate(
        [annotations[:, :, :4], area_b[:, :, None], annotations[:, :, 4:5]],
        axis=2)

    sums = pl.pallas_call(
        _focal_kernel,
        grid=(_B, _NA),
        in_specs=[
            pl.BlockSpec(memory_space=pltpu.SMEM),
            pl.BlockSpec((1, _BA, _C), lambda b, a: (b, a, 0)),
            pl.BlockSpec((1, 4, _LR, 128), lambda b, a: (b, 0, a, 0)),
            pl.BlockSpec((4, _LR, 128), lambda b, a: (0, a, 0)),
        ],
        out_specs=pl.BlockSpec((1, 1, 128), lambda b, a: (b, 0, 0)),
        out_shape=jax.ShapeDtypeStruct((_B, 1, 128), jnp.float32),
    )(ann, classifications, reg, anc)
    clf_sum = sums[:, 0, 0]
    reg_sum = sums[:, 0, 1]
    npos = sums[:, 0, 2]
    clf = clf_sum / jnp.clip(npos, 1.0, None)
    reg_l = reg_sum / jnp.clip(npos * 4.0, 1.0, None)
    return jnp.concatenate([jnp.mean(clf, keepdims=True),
                            jnp.mean(reg_l, keepdims=True)])


# speedup vs baseline: 6.1381x; 1.2402x over previous
import jax
import jax.numpy as jnp
from jax.experimental import pallas as pl
from jax.experimental.pallas import tpu as pltpu

_B, _A, _C, _M = 4, 65536, 80, 32
_ALPHA, _EPS = 0.25, 1e-4
_BA = 4096
_NA2 = _A // (2 * _BA)
_LR = _BA // 128


def _half_block(ann_ref, cls_ref, reg_ref, anc_ref, bi):
    ax1 = anc_ref[0]
    ay1 = anc_ref[1]
    ax2 = anc_ref[2]
    ay2 = anc_ref[3]
    area_a = (ax2 - ax1) * (ay2 - ay1)

    def iou_of(m):
        bx1 = ann_ref[bi, m, 0]
        by1 = ann_ref[bi, m, 1]
        bx2 = ann_ref[bi, m, 2]
        by2 = ann_ref[bi, m, 3]
        area_b = ann_ref[bi, m, 4]
        iw = jnp.clip(jnp.minimum(ax2, bx2) - jnp.maximum(ax1, bx1), 0.0, None)
        ih = jnp.clip(jnp.minimum(ay2, by2) - jnp.maximum(ay1, by1), 0.0, None)
        inter = iw * ih
        ua = jnp.clip(area_a + area_b - inter, 1e-8, None)
        return inter / ua

    best = iou_of(0)
    gx1 = jnp.full_like(best, ann_ref[bi, 0, 0])
    gy1 = jnp.full_like(best, ann_ref[bi, 0, 1])
    gx2 = jnp.full_like(best, ann_ref[bi, 0, 2])
    gy2 = jnp.full_like(best, ann_ref[bi, 0, 3])
    glab = jnp.full_like(best, ann_ref[bi, 0, 5])
    for m in range(1, _M):
        iou_m = iou_of(m)
        upd = iou_m > best
        best = jnp.where(upd, iou_m, best)
        gx1 = jnp.where(upd, ann_ref[bi, m, 0], gx1)
        gy1 = jnp.where(upd, ann_ref[bi, m, 1], gy1)
        gx2 = jnp.where(upd, ann_ref[bi, m, 2], gx2)
        gy2 = jnp.where(upd, ann_ref[bi, m, 3], gy2)
        glab = jnp.where(upd, ann_ref[bi, m, 5], glab)

    pos = best > 0.1
    posf = pos.astype(jnp.float32)
    valf = jnp.where(best <= 0.01, 1.0, posf)

    aw = ax2 - ax1
    ah = ay2 - ay1
    acx = ax1 + 0.5 * aw
    acy = ay1 + 0.5 * ah
    gwr = gx2 - gx1
    ghr = gy2 - gy1
    gcx = gx1 + 0.5 * gwr
    gcy = gy1 + 0.5 * ghr
    gw = jnp.clip(gwr, 1.0, None)
    gh = jnp.clip(ghr, 1.0, None)
    tdx = ((gcx - acx) / aw) / 0.1
    tdy = ((gcy - acy) / ah) / 0.1
    tdw = jnp.log(gw / aw) / 0.2
    tdh = jnp.log(gh / ah) / 0.2
    rl = jnp.zeros_like(best)
    for k, tk in enumerate((tdx, tdy, tdw, tdh)):
        d = jnp.abs(tk - reg_ref[0, k])
        rl = rl + jnp.where(d < 1.0 / 9.0, 0.5 * 9.0 * (d * d), d - 0.5 / 9.0)
    reg_part = jnp.sum(posf * rl)
    npos_part = jnp.sum(posf)

    cls_t = jnp.clip(cls_ref[0].T, _EPS, 1.0 - _EPS)
    term0 = (1.0 - _ALPHA) * (cls_t * cls_t) * (-jnp.log(1.0 - cls_t))
    s_row = jnp.sum(term0, axis=0)

    glab_row = jnp.reshape(glab, (1, _BA)).astype(jnp.int32)
    sub = jax.lax.broadcasted_iota(jnp.int32, (_C, _BA), 0)
    cls_sel = jnp.sum(jnp.where(sub == glab_row, cls_t, 0.0), axis=0)
    t0s = (1.0 - _ALPHA) * (cls_sel * cls_sel) * (-jnp.log(1.0 - cls_sel))
    t1s = _ALPHA * ((1.0 - cls_sel) * (1.0 - cls_sel)) * (-jnp.log(cls_sel))
    valf_row = jnp.reshape(valf, (_BA,))
    posf_row = jnp.reshape(posf, (_BA,))
    clf_part = jnp.sum(valf_row * s_row + posf_row * (t1s - t0s))
    return clf_part, reg_part, npos_part


def _focal_kernel(ann_ref, cls0_ref, cls1_ref, reg0_ref, reg1_ref,
                  anc0_ref, anc1_ref, out_ref):
    ai = pl.program_id(0)
    bi = pl.program_id(1)

    @pl.when(jnp.logical_and(ai == 0, bi == 0))
    def _init():
        out_ref[...] = jnp.zeros_like(out_ref)

    c0, r0, n0 = _half_block(ann_ref, cls0_ref, reg0_ref, anc0_ref, bi)
    c1, r1, n1 = _half_block(ann_ref, cls1_ref, reg1_ref, anc1_ref, bi)
    clf_part = c0 + c1
    reg_part = r0 + r1
    npos_part = n0 + n1

    lane = jax.lax.broadcasted_iota(jnp.int32, (_B, 1, 128), 2)
    brow = jax.lax.broadcasted_iota(jnp.int32, (_B, 1, 128), 0)
    vec = (jnp.where(lane == 0, clf_part, 0.0)
           + jnp.where(lane == 1, reg_part, 0.0)
           + jnp.where(lane == 2, npos_part, 0.0))
    out_ref[...] += jnp.where(brow == bi, vec, 0.0)


def kernel(classifications, regressions, anchors, annotations):
    anc = jnp.transpose(anchors[0]).reshape(4, _A // 128, 128)
    reg = jnp.transpose(regressions, (0, 2, 1)).reshape(_B, 4, _A // 128, 128)
    area_b = ((annotations[:, :, 2] - annotations[:, :, 0])
              * (annotations[:, :, 3] - annotations[:, :, 1]))
    ann = jnp.concatenate(
        [annotations[:, :, :4], area_b[:, :, None], annotations[:, :, 4:5]],
        axis=2)

    sums = pl.pallas_call(
        _focal_kernel,
        grid=(_NA2, _B),
        in_specs=[
            pl.BlockSpec(memory_space=pltpu.SMEM),
            pl.BlockSpec((1, _BA, _C), lambda a, b: (b, 2 * a, 0)),
            pl.BlockSpec((1, _BA, _C), lambda a, b: (b, 2 * a + 1, 0)),
            pl.BlockSpec((1, 4, _LR, 128), lambda a, b: (b, 0, 2 * a, 0)),
            pl.BlockSpec((1, 4, _LR, 128), lambda a, b: (b, 0, 2 * a + 1, 0)),
            pl.BlockSpec((4, _LR, 128), lambda a, b: (0, 2 * a, 0)),
            pl.BlockSpec((4, _LR, 128), lambda a, b: (0, 2 * a + 1, 0)),
        ],
        out_specs=pl.BlockSpec((_B, 1, 128), lambda a, b: (0, 0, 0)),
        out_shape=jax.ShapeDtypeStruct((_B, 1, 128), jnp.float32),
    )(ann, classifications, classifications, reg, reg, anc, anc)
    clf_sum = sums[:, 0, 0]
    reg_sum = sums[:, 0, 1]
    npos = sums[:, 0, 2]
    clf = clf_sum / jnp.clip(npos, 1.0, None)
    reg_l = reg_sum / jnp.clip(npos * 4.0, 1.0, None)
    return jnp.concatenate([jnp.mean(clf, keepdims=True),
                            jnp.mean(reg_l, keepdims=True)])
